# Initial kernel scaffold; baseline (speedup 1.0000x reference)
import jax
import jax.numpy as jnp
from jax import lax
from jax.experimental import pallas as pl
from jax.experimental.pallas import tpu as pltpu
from jax.experimental.pallas import tpu_sc as plsc

NUM_EMB = 1000000
DIM = 32
B_TOTAL = 16384 * 50
NC, NS = 2, 16
NW = NC * NS
IDX_PER_GROUP = 128
GROUPS_PER_W = B_TOTAL // (NW * IDX_PER_GROUP)
G_PER_CHUNK = 10
CHUNK_ROWS = G_PER_CHUNK * IDX_PER_GROUP
N_CHUNKS = GROUPS_PER_W // G_PER_CHUNK
ROWS_PER_W = GROUPS_PER_W * IDX_PER_GROUP


def _emb_body(idx_hbm, table_hbm, out_hbm, idx_v, buf0, buf1, gsem0, gsem1):
    c = lax.axis_index("c")
    s = lax.axis_index("s")
    wid = s * NC + c
    gbase = wid * GROUPS_PER_W
    rbase = wid * ROWS_PER_W

    pltpu.sync_copy(idx_hbm.at[pl.ds(gbase, GROUPS_PER_W)], idx_v)

    bufs = (buf0, buf1)
    gsems = (gsem0, gsem1)

    def fire(chunk, b):
        for j in range(G_PER_CHUNK):
            pltpu.async_copy(
                table_hbm.at[idx_v.at[chunk * G_PER_CHUNK + j]],
                bufs[b].at[pl.ds(j * IDX_PER_GROUP, IDX_PER_GROUP)],
                gsems[b],
            )

    def drain(b):
        pltpu.make_async_copy(
            out_hbm.at[pl.ds(0, CHUNK_ROWS)], bufs[b], gsems[b]
        ).wait()

    fire(0, 0)
    fire(1, 1)

    def step(it, carry):
        chunk = it * 2
        for b in range(2):
            cc = chunk + b
            drain(b)
            pltpu.sync_copy(
                bufs[b], out_hbm.at[pl.ds(rbase + cc * CHUNK_ROWS, CHUNK_ROWS)]
            )

            @pl.when(cc + 2 < N_CHUNKS)
            def _():
                fire(cc + 2, b)

        return carry

    lax.fori_loop(0, N_CHUNKS // 2, step, 0)


@jax.jit
def _emb_lookup(idx_flat, weight):
    mesh = plsc.VectorSubcoreMesh(
        core_axis_name="c", subcore_axis_name="s", num_cores=NC, num_subcores=NS
    )
    f = pl.kernel(
        _emb_body,
        out_type=jax.ShapeDtypeStruct((B_TOTAL, DIM), jnp.float32),
        mesh=mesh,
        scratch_types=[
            pltpu.VMEM((GROUPS_PER_W, IDX_PER_GROUP), jnp.int32),
            pltpu.VMEM((CHUNK_ROWS, DIM), jnp.float32),
            pltpu.VMEM((CHUNK_ROWS, DIM), jnp.float32),
            pltpu.SemaphoreType.DMA,
            pltpu.SemaphoreType.DMA,
        ],
    )
    return f(idx_flat, weight)


def kernel(input_, weight):
    idx_flat = input_.astype(jnp.int32).reshape(B_TOTAL // IDX_PER_GROUP, IDX_PER_GROUP)
    out = _emb_lookup(idx_flat, weight)
    return out.reshape(input_.shape[0], input_.shape[1], DIM)


# speedup vs baseline: 1.1134x; 1.1134x over previous
import jax
import jax.numpy as jnp
from jax import lax
from jax.experimental import pallas as pl
from jax.experimental.pallas import tpu as pltpu
from jax.experimental.pallas import tpu_sc as plsc

NUM_EMB = 1000000
DIM = 32
B_TOTAL = 16384 * 50
NC, NS = 2, 16
NW = NC * NS
IDX_PER_GROUP = 128
GROUPS_PER_W = B_TOTAL // (NW * IDX_PER_GROUP)
G_PER_CHUNK = 10
CHUNK_ROWS = G_PER_CHUNK * IDX_PER_GROUP
N_CHUNKS = GROUPS_PER_W // G_PER_CHUNK
ROWS_PER_W = GROUPS_PER_W * IDX_PER_GROUP


def _emb_body(idx_hbm, table_hbm, out_hbm, idx_v, buf0, buf1, gsem0, gsem1):
    c = lax.axis_index("c")
    s = lax.axis_index("s")
    wid = s * NC + c
    gbase = wid * GROUPS_PER_W
    rbase = wid * ROWS_PER_W

    pltpu.sync_copy(idx_hbm.at[pl.ds(gbase, GROUPS_PER_W)], idx_v)

    bufs = (buf0, buf1)
    gsems = (gsem0, gsem1)

    def fire(chunk, b):
        for j in range(G_PER_CHUNK):
            pltpu.async_copy(
                table_hbm.at[idx_v.at[chunk * G_PER_CHUNK + j]],
                bufs[b].at[pl.ds(j * IDX_PER_GROUP, IDX_PER_GROUP)],
                gsems[b],
            )

    def drain(b):
        pltpu.make_async_copy(
            out_hbm.at[pl.ds(0, CHUNK_ROWS)], bufs[b], gsems[b]
        ).wait()

    fire(0, 0)
    fire(1, 1)

    def step(it, carry):
        chunk = it * 2
        for b in range(2):
            cc = chunk + b
            drain(b)
            pltpu.sync_copy(
                bufs[b], out_hbm.at[pl.ds(rbase + cc * CHUNK_ROWS, CHUNK_ROWS)]
            )

            @pl.when(cc + 2 < N_CHUNKS)
            def _():
                fire(cc + 2, b)

        return carry

    lax.fori_loop(0, N_CHUNKS // 2, step, 0)


@jax.jit
def _emb_lookup(idx_flat, weight):
    mesh = plsc.VectorSubcoreMesh(
        core_axis_name="c", subcore_axis_name="s", num_cores=NC, num_subcores=NS
    )
    f = pl.kernel(
        _emb_body,
        out_type=jax.ShapeDtypeStruct((B_TOTAL, DIM), jnp.float32),
        mesh=mesh,
        scratch_types=[
            pltpu.VMEM((GROUPS_PER_W, IDX_PER_GROUP), jnp.int32),
            pltpu.VMEM((CHUNK_ROWS, DIM), jnp.float32),
            pltpu.VMEM((CHUNK_ROWS, DIM), jnp.float32),
            pltpu.SemaphoreType.DMA,
            pltpu.SemaphoreType.DMA,
        ],
        compiler_params=pltpu.CompilerParams(use_tc_tiling_on_sc=False),
    )
    return f(idx_flat, weight)


def kernel(input_, weight):
    idx_flat = input_.astype(jnp.int32).reshape(B_TOTAL // IDX_PER_GROUP, IDX_PER_GROUP)
    out = _emb_lookup(idx_flat, weight)
    return out.reshape(input_.shape[0], input_.shape[1], DIM)
